# Initial kernel scaffold; baseline (speedup 1.0000x reference)
import jax
import jax.numpy as jnp
from jax.experimental import pallas as pl


def kernel(x, sn, node, params):
    raise NotImplementedError("write your pallas kernel here")


# speedup vs baseline: 29.3838x; 29.3838x over previous
import functools

import jax
import jax.numpy as jnp
from jax.experimental import pallas as pl
from jax.experimental.pallas import tpu as pltpu

F32 = jnp.float32
KS = 3
KNN = 16
EPS = 1e-5
SEG = 128


def _oh(idx_vec, n, m):
    io = jax.lax.broadcasted_iota(jnp.int32, (n, m), 1)
    return (io == idx_vec[:, None]).astype(F32)


def _dt(a, b):
    return jax.lax.dot_general(a, b, (((1,), (1,)), ((), ())),
                               preferred_element_type=F32)


def _mm(a, b):
    return jnp.dot(a, b, preferred_element_type=F32)


def _assign_k(x_ref, node_ref, idx_ref, cnt_ref, sums_ref, *, pn, m):
    t = pl.program_id(1)

    @pl.when(t == 0)
    def _():
        cnt_ref[...] = jnp.zeros_like(cnt_ref)
        sums_ref[...] = jnp.zeros_like(sums_ref)

    x = x_ref[0]
    nd = node_ref[0]
    d = ((x[0][:, None] - nd[0][None, :]) ** 2
         + (x[1][:, None] - nd[1][None, :]) ** 2) \
        + (x[2][:, None] - nd[2][None, :]) ** 2
    io = jax.lax.broadcasted_iota(jnp.int32, (pn, m), 1)
    cnt = jnp.zeros((1, m), F32)
    sm = jnp.zeros((3, m), F32)
    for j in range(KS):
        dmin = jnp.min(d, axis=1, keepdims=True)
        idxj = jnp.min(jnp.where(d == dmin, io, m), axis=1)
        idx_ref[0, j, :] = idxj
        ohj = (io == idxj[:, None]).astype(F32)
        cnt = cnt + jnp.sum(ohj, axis=0, keepdims=True)
        sm = sm + jax.lax.dot_general(x, ohj, (((1,), (0,)), ((), ())),
                                      preferred_element_type=F32)
        if j < KS - 1:
            d = jnp.where(io == idxj[:, None], 1e30, d)
    cnt_ref[0] += cnt
    sums_ref[0] += sm


def _scale_shift(st, g_ref, be_ref, cnt):
    tot1 = jnp.sum(st[:, 0, :], axis=0)
    tot2 = jnp.sum(st[:, 1, :], axis=0)
    mean = tot1 / cnt
    var = tot2 / cnt - mean * mean
    scale = g_ref[...][:, 0] / jnp.sqrt(var + EPS)
    shift = be_ref[...][:, 0] - mean * scale
    return scale[:, None], shift[:, None]


def _l1_k(x_ref, sn_ref, idx_ref, sums_ref, cnt_ref, w_ref, b_ref,
          y_ref, st_ref, *, pn, m, cout):
    t = pl.program_id(1)

    @pl.when(t == 0)
    def _():
        st_ref[...] = jnp.zeros_like(st_ref)

    cm = sums_ref[0] / (cnt_ref[0] + 1e-5)
    W = w_ref[...]
    bb = b_ref[...]
    x = x_ref[0]
    sn = sn_ref[0]
    s1 = jnp.zeros((cout,), F32)
    s2 = jnp.zeros((cout,), F32)
    for k in range(KS):
        oh = _oh(idx_ref[0, k, :], pn, m)
        cen = _dt(cm, oh)
        xa = jnp.concatenate([x - cen, sn], axis=0)
        y = _mm(W, xa) + bb
        y_ref[0, :, k, :] = y
        s1 += jnp.sum(y, axis=1)
        s2 += jnp.sum(y * y, axis=1)
    st_ref[0, 0, :] += s1
    st_ref[0, 1, :] += s2


def _lmid_k(y_ref, st_ref, g_ref, be_ref, w_ref, b_ref, o_ref, so_ref,
            *, cnt, cout):
    t = pl.program_id(1)

    @pl.when(t == 0)
    def _():
        so_ref[...] = jnp.zeros_like(so_ref)

    scale, shift = _scale_shift(st_ref[...], g_ref, be_ref, cnt)
    W = w_ref[...]
    bb = b_ref[...]
    s1 = jnp.zeros((cout,), F32)
    s2 = jnp.zeros((cout,), F32)
    for k in range(KS):
        h = jnp.maximum(y_ref[0, :, k, :] * scale + shift, 0.0)
        z = _mm(W, h) + bb
        o_ref[0, :, k, :] = z
        s1 += jnp.sum(z, axis=1)
        s2 += jnp.sum(z * z, axis=1)
    so_ref[0, 0, :] += s1
    so_ref[0, 1, :] += s2


def _l4_k(f_ref, idx_ref, fm_ref, w_ref, b_ref, o_ref, so_ref,
          *, pn, m, cin, cout):
    t = pl.program_id(1)

    @pl.when(t == 0)
    def _():
        so_ref[...] = jnp.zeros_like(so_ref)

    W = w_ref[...]
    Wf = W[:, :cin]
    Ws = W[:, cin:]
    bb = b_ref[...]
    fm = fm_ref[0]
    s1 = jnp.zeros((cout,), F32)
    s2 = jnp.zeros((cout,), F32)
    for k in range(KS):
        f = f_ref[0, :, k, :]
        oh = _oh(idx_ref[0, k, :], pn, m)
        scat = _dt(fm, oh)
        z = _mm(Wf, f) + _mm(Ws, scat) + bb
        o_ref[0, :, k, :] = z
        s1 += jnp.sum(z, axis=1)
        s2 += jnp.sum(z * z, axis=1)
    so_ref[0, 0, :] += s1
    so_ref[0, 1, :] += s2


def _segmax_ref(src_ref, idx_ref, k, fm, pn, m):
    def body(i, fmc):
        blk = src_ref[0, :, k, pl.ds(i * SEG, SEG)]
        iv = idx_ref[0, k, pl.ds(i * SEG, SEG)]
        ohb = _oh(iv, SEG, m)
        contrib = jnp.max(blk[:, :, None] * ohb[None, :, :], axis=1)
        return jnp.maximum(fmc, contrib)

    return jax.lax.fori_loop(0, pn // SEG, body, fm)


def _fmax_store_k(y_ref, st_ref, g_ref, be_ref, idx_ref, f_ref, fm_ref,
                  *, cnt, pn, m):
    t = pl.program_id(1)

    @pl.when(t == 0)
    def _():
        fm_ref[...] = jnp.zeros_like(fm_ref)

    scale, shift = _scale_shift(st_ref[...], g_ref, be_ref, cnt)
    fm = fm_ref[0]
    for k in range(KS):
        f = jnp.maximum(y_ref[0, :, k, :] * scale + shift, 0.0)
        f_ref[0, :, k, :] = f
        fm = _segmax_ref(f_ref, idx_ref, k, fm, pn, m)
    fm_ref[0] = fm


def _fmax_only_k(y_ref, st_ref, g_ref, be_ref, idx_ref, fm_ref, f_scr,
                 *, cnt, pn, m):
    t = pl.program_id(1)

    @pl.when(t == 0)
    def _():
        fm_ref[...] = jnp.zeros_like(fm_ref)

    scale, shift = _scale_shift(st_ref[...], g_ref, be_ref, cnt)
    fm = fm_ref[0]
    for k in range(KS):
        f_scr[0, :, k, :] = jnp.maximum(y_ref[0, :, k, :] * scale + shift,
                                        0.0)
    for k in range(KS):
        fm = _segmax_ref(f_scr, idx_ref, k, fm, pn, m)
    fm_ref[0] = fm


def _norm_relu_multi(zs, g_ref, be_ref, cnt):
    s1 = zs[0].sum(axis=1)
    for z in zs[1:]:
        s1 = s1 + z.sum(axis=1)
    mean = s1 / cnt
    s2 = ((zs[0] - mean[:, None]) ** 2).sum(axis=1)
    for z in zs[1:]:
        s2 = s2 + ((z - mean[:, None]) ** 2).sum(axis=1)
    var = s2 / cnt
    scale = (g_ref[...][:, 0] / jnp.sqrt(var + EPS))[:, None]
    shift = (be_ref[...][:, 0])[:, None] - mean[:, None] * scale
    return [jnp.maximum(z * scale + shift, 0.0) for z in zs]


def _knn_k(sums_ref, cnt_ref, fm2_ref,
           w1_ref, b1_ref, g1_ref, be1_ref,
           w2_ref, b2_ref, g2_ref, be2_ref,
           w3_ref, b3_ref, g3_ref, be3_ref,
           g_out_ref, *, nb, m):
    z1s = []
    for b in range(nb):
        cm = sums_ref[b] / (cnt_ref[b] + 1e-5)
        fm = fm2_ref[b]
        d = ((cm[0][:, None] - cm[0][None, :]) ** 2
             + (cm[1][:, None] - cm[1][None, :]) ** 2) \
            + (cm[2][:, None] - cm[2][None, :]) ** 2
        io = jax.lax.broadcasted_iota(jnp.int32, (m, m), 1)
        cols = []
        for j in range(KNN):
            dmin = jnp.min(d, axis=1, keepdims=True)
            idxj = jnp.min(jnp.where(d == dmin, io, m), axis=1)
            if j < KNN - 1:
                d = jnp.where(io == idxj[:, None], 1e30, d)
            ohj = (io == idxj[:, None]).astype(F32)
            nbc = _dt(cm, ohj)
            nbf = _dt(fm, ohj)
            cols.append(jnp.concatenate([nbc - cm, nbf], axis=0))
        in1 = jnp.concatenate(cols, axis=1)
        z1s.append(_mm(w1_ref[...], in1) + b1_ref[...])
    cntn = nb * KNN * m
    h1 = _norm_relu_multi(z1s, g1_ref, be1_ref, cntn)
    z2s = [_mm(w2_ref[...], h) + b2_ref[...] for h in h1]
    h2 = _norm_relu_multi(z2s, g2_ref, be2_ref, cntn)
    z3s = [_mm(w3_ref[...], h) + b3_ref[...] for h in h2]
    h3 = _norm_relu_multi(z3s, g3_ref, be3_ref, cntn)
    for b in range(nb):
        gm = h3[b][:, 0:m]
        for j in range(1, KNN):
            gm = jnp.maximum(gm, h3[b][:, j * m:(j + 1) * m])
        g_out_ref[b] = gm


def _tail_k(g_ref, fm2_ref, sums_ref, cnt_ref,
            ws1_ref, bs1_ref, gs1_ref, bes1_ref,
            ws2_ref, bs2_ref, gs2_ref, bes2_ref,
            wm1_ref, bm1_ref, gm1_ref, bem1_ref,
            wm2_ref, bm2_ref, gm2_ref, bem2_ref,
            wm3_ref, bm3_ref,
            cm_ref, kp_ref, sg_ref, *, nb, m, c2):
    cntn = nb * m
    z1s = [_mm(ws1_ref[...], g_ref[b]) + bs1_ref[...] for b in range(nb)]
    h1 = _norm_relu_multi(z1s, gs1_ref, bes1_ref, cntn)
    z2s = [_mm(ws2_ref[...], h) + bs2_ref[...] for h in h1]
    h2 = _norm_relu_multi(z2s, gs2_ref, bes2_ref, cntn)
    wm1 = wm1_ref[...]
    z3s = [_mm(wm1[:, :c2], fm2_ref[b]) + _mm(wm1[:, c2:], h2[b])
           + bm1_ref[...] for b in range(nb)]
    h3 = _norm_relu_multi(z3s, gm1_ref, bem1_ref, cntn)
    z4s = [_mm(wm2_ref[...], h) + bm2_ref[...] for h in h3]
    h4 = _norm_relu_multi(z4s, gm2_ref, bem2_ref, cntn)
    for b in range(nb):
        ks = _mm(wm3_ref[...], h4[b]) + bm3_ref[...]
        cm = sums_ref[b] / (cnt_ref[b] + 1e-5)
        cm_ref[b] = cm
        kp_ref[b] = ks[0:3, :] + cm
        s = ks[3, :]
        sg_ref[b] = jnp.maximum(s, 0.0) + jnp.log1p(jnp.exp(-jnp.abs(s))) \
            + 0.001


def _pb(p):
    return p["W"], p["b"].reshape(-1, 1), p["gamma"].reshape(-1, 1), \
        p["beta"].reshape(-1, 1)


def kernel(x, sn, node, params):
    nb, _, n = x.shape
    m = node.shape[2]
    pn = min(1024, n)
    t = n // pn
    cntN = nb * KS * n

    def full(shape):
        nd = len(shape)
        return pl.BlockSpec(shape, lambda b, i: (0,) * nd)

    def perb(shape):
        nd = len(shape) - 1
        return pl.BlockSpec(shape, lambda b, i: (b,) + (0,) * nd)

    def tile(shape, ax):
        def imap(b, i, _ax=ax):
            out = [0] * len(shape)
            out[0] = b
            out[_ax] = i
            return tuple(out)
        return pl.BlockSpec(shape, imap)

    idx, cnt, sums = pl.pallas_call(
        functools.partial(_assign_k, pn=pn, m=m),
        grid=(nb, t),
        in_specs=[tile((1, 3, pn), 2), perb((1, 3, m))],
        out_specs=[tile((1, KS, pn), 2), perb((1, 1, m)), perb((1, 3, m))],
        out_shape=[jax.ShapeDtypeStruct((nb, KS, n), jnp.int32),
                   jax.ShapeDtypeStruct((nb, 1, m), F32),
                   jax.ShapeDtypeStruct((nb, 3, m), F32)],
    )(x, node)

    w1, b1, g1, be1 = _pb(params["first_pn"][0])
    c = w1.shape[0]
    y, st = pl.pallas_call(
        functools.partial(_l1_k, pn=pn, m=m, cout=c),
        grid=(nb, t),
        in_specs=[tile((1, 3, pn), 2), tile((1, 3, pn), 2),
                  tile((1, KS, pn), 2), perb((1, 3, m)), perb((1, 1, m)),
                  full(w1.shape), full(b1.shape)],
        out_specs=[tile((1, c, KS, pn), 3), perb((1, 2, c))],
        out_shape=[jax.ShapeDtypeStruct((nb, c, KS, n), F32),
                   jax.ShapeDtypeStruct((nb, 2, c), F32)],
    )(x, sn, idx, sums, cnt, w1, b1)

    def mid_layer(y, st, p, gprev, beprev):
        w, b, g, be = _pb(p)
        cin = w.shape[1]
        cout = w.shape[0]
        return pl.pallas_call(
            functools.partial(_lmid_k, cnt=float(cntN), cout=cout),
            grid=(nb, t),
            in_specs=[tile((1, cin, KS, pn), 3), full((nb, 2, cin)),
                      full(gprev.shape), full(beprev.shape),
                      full(w.shape), full(b.shape)],
            out_specs=[tile((1, cout, KS, pn), 3), perb((1, 2, cout))],
            out_shape=[jax.ShapeDtypeStruct((nb, cout, KS, n), F32),
                       jax.ShapeDtypeStruct((nb, 2, cout), F32)],
        )(y, st, gprev, beprev, w, b), (g, be)

    _, _, g_c, be_c = _pb(params["first_pn"][0])
    (y, st), (g_c, be_c) = mid_layer(y, st, params["first_pn"][1], g_c, be_c)
    (y, st), (g_c, be_c) = mid_layer(y, st, params["first_pn"][2], g_c, be_c)

    f1, f1m = pl.pallas_call(
        functools.partial(_fmax_store_k, cnt=float(cntN), pn=pn, m=m),
        grid=(nb, t),
        in_specs=[tile((1, c, KS, pn), 3), full((nb, 2, c)),
                  full(g_c.shape), full(be_c.shape), tile((1, KS, pn), 2)],
        out_specs=[tile((1, c, KS, pn), 3), perb((1, c, m))],
        out_shape=[jax.ShapeDtypeStruct((nb, c, KS, n), F32),
                   jax.ShapeDtypeStruct((nb, c, m), F32)],
    )(y, st, g_c, be_c, idx)

    w4, b4, g4, be4 = _pb(params["second_pn"][0])
    c2 = w4.shape[0]
    y, st = pl.pallas_call(
        functools.partial(_l4_k, pn=pn, m=m, cin=c, cout=c2),
        grid=(nb, t),
        in_specs=[tile((1, c, KS, pn), 3), tile((1, KS, pn), 2),
                  perb((1, c, m)), full(w4.shape), full(b4.shape)],
        out_specs=[tile((1, c2, KS, pn), 3), perb((1, 2, c2))],
        out_shape=[jax.ShapeDtypeStruct((nb, c2, KS, n), F32),
                   jax.ShapeDtypeStruct((nb, 2, c2), F32)],
    )(f1, idx, f1m, w4, b4)

    (y, st), (g_c, be_c) = mid_layer(y, st, params["second_pn"][1], g4, be4)

    f2m = pl.pallas_call(
        functools.partial(_fmax_only_k, cnt=float(cntN), pn=pn, m=m),
        grid=(nb, t),
        in_specs=[tile((1, c2, KS, pn), 3), full((nb, 2, c2)),
                  full(g_c.shape), full(be_c.shape), tile((1, KS, pn), 2)],
        out_specs=perb((1, c2, m)),
        out_shape=jax.ShapeDtypeStruct((nb, c2, m), F32),
        scratch_shapes=[pltpu.VMEM((1, c2, KS, pn), F32)],
    )(y, st, g_c, be_c, idx)

    kf = [_pb(p) for p in params["knn_first"]]
    ck = kf[0][0].shape[0]
    g_nodes = pl.pallas_call(
        functools.partial(_knn_k, nb=nb, m=m),
        out_shape=jax.ShapeDtypeStruct((nb, ck, m), F32),
    )(sums, cnt, f2m, *kf[0], *kf[1], *kf[2])

    ksnd = [_pb(p) for p in params["knn_second"]]
    m1 = _pb(params["mlp1"][0])
    m2 = _pb(params["mlp2"][0])
    w3p = params["mlp3"][0]["W"]
    b3p = params["mlp3"][0]["b"].reshape(-1, 1)
    cm_out, kp, sg = pl.pallas_call(
        functools.partial(_tail_k, nb=nb, m=m, c2=c2),
        out_shape=[jax.ShapeDtypeStruct((nb, 3, m), F32),
                   jax.ShapeDtypeStruct((nb, 3, m), F32),
                   jax.ShapeDtypeStruct((nb, m), F32)],
    )(g_nodes, f2m, sums, cnt, *ksnd[0], *ksnd[1], *m1, *m2, w3p, b3p)

    return (cm_out, kp, sg)


# speedup vs baseline: 58.6599x; 1.9963x over previous
import functools

import jax
import jax.numpy as jnp
from jax import lax
from jax.experimental import pallas as pl
from jax.experimental.pallas import tpu as pltpu
from jax.experimental.pallas import tpu_sc as plsc

F32 = jnp.float32
KS = 3
KNN = 16
EPS = 1e-5
SEG = 128


def _oh(idx_vec, n, m):
    io = jax.lax.broadcasted_iota(jnp.int32, (n, m), 1)
    return (io == idx_vec[:, None]).astype(F32)


def _dt(a, b):
    return jax.lax.dot_general(a, b, (((1,), (1,)), ((), ())),
                               preferred_element_type=F32)


def _mm(a, b):
    return jnp.dot(a, b, preferred_element_type=F32)


def _assign_k(x_ref, node_ref, idx_ref, cnt_ref, sums_ref, *, pn, m):
    t = pl.program_id(1)

    @pl.when(t == 0)
    def _():
        cnt_ref[...] = jnp.zeros_like(cnt_ref)
        sums_ref[...] = jnp.zeros_like(sums_ref)

    x = x_ref[0]
    nd = node_ref[0]
    d = ((x[0][:, None] - nd[0][None, :]) ** 2
         + (x[1][:, None] - nd[1][None, :]) ** 2) \
        + (x[2][:, None] - nd[2][None, :]) ** 2
    io = jax.lax.broadcasted_iota(jnp.int32, (pn, m), 1)
    cnt = jnp.zeros((1, m), F32)
    sm = jnp.zeros((3, m), F32)
    for j in range(KS):
        dmin = jnp.min(d, axis=1, keepdims=True)
        idxj = jnp.min(jnp.where(d == dmin, io, m), axis=1)
        idx_ref[0, j, :] = idxj
        ohj = (io == idxj[:, None]).astype(F32)
        cnt = cnt + jnp.sum(ohj, axis=0, keepdims=True)
        sm = sm + jax.lax.dot_general(x, ohj, (((1,), (0,)), ((), ())),
                                      preferred_element_type=F32)
        if j < KS - 1:
            d = jnp.where(io == idxj[:, None], 1e30, d)
    cnt_ref[0] += cnt
    sums_ref[0] += sm


def _scale_shift(st, g_ref, be_ref, cnt):
    tot1 = jnp.sum(st[:, 0, :], axis=0)
    tot2 = jnp.sum(st[:, 1, :], axis=0)
    mean = tot1 / cnt
    var = tot2 / cnt - mean * mean
    scale = g_ref[...][:, 0] / jnp.sqrt(var + EPS)
    shift = be_ref[...][:, 0] - mean * scale
    return scale[:, None], shift[:, None]


def _l1_k(x_ref, sn_ref, idx_ref, sums_ref, cnt_ref, w_ref, b_ref,
          y_ref, st_ref, *, pn, m, cout):
    t = pl.program_id(1)

    @pl.when(t == 0)
    def _():
        st_ref[...] = jnp.zeros_like(st_ref)

    cm = sums_ref[0] / (cnt_ref[0] + 1e-5)
    W = w_ref[...]
    bb = b_ref[...]
    x = x_ref[0]
    sn = sn_ref[0]
    s1 = jnp.zeros((cout,), F32)
    s2 = jnp.zeros((cout,), F32)
    for k in range(KS):
        oh = _oh(idx_ref[0, k, :], pn, m)
        cen = _dt(cm, oh)
        xa = jnp.concatenate([x - cen, sn], axis=0)
        y = _mm(W, xa) + bb
        y_ref[0, :, k, :] = y
        s1 += jnp.sum(y, axis=1)
        s2 += jnp.sum(y * y, axis=1)
    st_ref[0, 0, :] += s1
    st_ref[0, 1, :] += s2


def _lmid_k(y_ref, st_ref, g_ref, be_ref, w_ref, b_ref, o_ref, so_ref,
            *, cnt, cout):
    t = pl.program_id(1)

    @pl.when(t == 0)
    def _():
        so_ref[...] = jnp.zeros_like(so_ref)

    scale, shift = _scale_shift(st_ref[...], g_ref, be_ref, cnt)
    W = w_ref[...]
    bb = b_ref[...]
    s1 = jnp.zeros((cout,), F32)
    s2 = jnp.zeros((cout,), F32)
    for k in range(KS):
        h = jnp.maximum(y_ref[0, :, k, :] * scale + shift, 0.0)
        z = _mm(W, h) + bb
        o_ref[0, :, k, :] = z
        s1 += jnp.sum(z, axis=1)
        s2 += jnp.sum(z * z, axis=1)
    so_ref[0, 0, :] += s1
    so_ref[0, 1, :] += s2


def _l4_k(f_ref, idx_ref, fm_ref, w_ref, b_ref, o_ref, so_ref,
          *, pn, m, cin, cout):
    t = pl.program_id(1)

    @pl.when(t == 0)
    def _():
        so_ref[...] = jnp.zeros_like(so_ref)

    W = w_ref[...]
    Wf = W[:, :cin]
    Ws = W[:, cin:]
    bb = b_ref[...]
    fm = jnp.max(fm_ref[0], axis=0)
    s1 = jnp.zeros((cout,), F32)
    s2 = jnp.zeros((cout,), F32)
    for k in range(KS):
        f = f_ref[0, :, k, :]
        oh = _oh(idx_ref[0, k, :], pn, m)
        scat = _dt(fm, oh)
        z = _mm(Wf, f) + _mm(Ws, scat) + bb
        o_ref[0, :, k, :] = z
        s1 += jnp.sum(z, axis=1)
        s2 += jnp.sum(z * z, axis=1)
    so_ref[0, 0, :] += s1
    so_ref[0, 1, :] += s2


def _normf_k(y_ref, st_ref, g_ref, be_ref, f_ref, *, cnt):
    scale, shift = _scale_shift(st_ref[...], g_ref, be_ref, cnt)
    for k in range(KS):
        f_ref[0, :, k, :] = jnp.maximum(y_ref[0, :, k, :] * scale + shift,
                                        0.0)


def _dg(v, i):
    return lax.gather(
        v, i[:, None],
        lax.GatherDimensionNumbers(offset_dims=(), collapsed_slice_dims=(0,),
                                   start_index_map=(0,)),
        (1,), mode=lax.GatherScatterMode.PROMISE_IN_BOUNDS)


def _sc_segmax(f, idx_flat, nb, c, n, m):
    nwb = 32 // nb
    ch = n // nwb
    mesh = plsc.VectorSubcoreMesh(core_axis_name="c", subcore_axis_name="s")

    @functools.partial(
        pl.kernel, mesh=mesh,
        compiler_params=pltpu.CompilerParams(needs_layout_passes=False),
        out_type=jax.ShapeDtypeStruct((nb * nwb * c * m,), F32),
        scratch_types=[
            pltpu.VMEM((1, c, 1, ch), F32),
            pltpu.VMEM((ch,), jnp.int32),
            pltpu.VMEM((c * m,), F32),
        ],
    )
    def k(f_hbm, idx_hbm, out_hbm, fv, iv, acc):
        ci = lax.axis_index("c")
        si = lax.axis_index("s")
        w = si * 2 + ci
        b = w // nwb
        ww = w % nwb
        zero = jnp.zeros((16,), F32)

        def zbody(i, _):
            acc[pl.ds(i * 16, 16)] = zero
            return 0

        lax.fori_loop(0, c * m // 16, zbody, 0)

        io = lax.iota(jnp.int32, 16)
        for kk in range(KS):
            pltpu.sync_copy(
                idx_hbm.at[pl.ds((b * KS + kk) * n + ww * ch, ch)], iv)
            pltpu.sync_copy(
                f_hbm.at[pl.ds(b, 1), :, pl.ds(kk, 1), pl.ds(ww * ch, ch)],
                fv)

            def gbody(g, _):
                idxv = iv[pl.ds(g * 16, 16)]
                keys, perm = plsc.sort_key_val(idxv, io)
                eqs = []
                for s in (1, 2, 4, 8):
                    sh = _dg(keys, jnp.maximum(io - s, 0))
                    eqs.append((sh == keys) & (io >= s))
                nxt = _dg(keys, jnp.minimum(io + 1, 15))
                last = (nxt != keys) | (io == 15)
                for cc in range(c):
                    v = fv[0, cc, 0, pl.ds(g * 16, 16)]
                    v = _dg(v, perm)
                    for ei, s in enumerate((1, 2, 4, 8)):
                        vs = _dg(v, jnp.maximum(io - s, 0))
                        v = jnp.where(eqs[ei], jnp.maximum(v, vs), v)
                    idxc = keys + cc * m
                    cur = plsc.load_gather(acc, [idxc])
                    plsc.store_scatter(acc, [idxc], jnp.maximum(cur, v),
                                       mask=last)
                return 0

            lax.fori_loop(0, ch // 16, gbody, 0)
        pltpu.sync_copy(acc, out_hbm.at[pl.ds((b * nwb + ww) * c * m,
                                              c * m)])

    return k(f, idx_flat).reshape(nb, nwb, c, m)


def _norm_relu_multi(zs, g_ref, be_ref, cnt):
    s1 = zs[0].sum(axis=1)
    for z in zs[1:]:
        s1 = s1 + z.sum(axis=1)
    mean = s1 / cnt
    s2 = ((zs[0] - mean[:, None]) ** 2).sum(axis=1)
    for z in zs[1:]:
        s2 = s2 + ((z - mean[:, None]) ** 2).sum(axis=1)
    var = s2 / cnt
    scale = (g_ref[...][:, 0] / jnp.sqrt(var + EPS))[:, None]
    shift = (be_ref[...][:, 0])[:, None] - mean[:, None] * scale
    return [jnp.maximum(z * scale + shift, 0.0) for z in zs]


def _knn_k(sums_ref, cnt_ref, fm2_ref,
           w1_ref, b1_ref, g1_ref, be1_ref,
           w2_ref, b2_ref, g2_ref, be2_ref,
           w3_ref, b3_ref, g3_ref, be3_ref,
           g_out_ref, fm2m_ref, *, nb, m):
    z1s = []
    for b in range(nb):
        cm = sums_ref[b] / (cnt_ref[b] + 1e-5)
        fm = jnp.max(fm2_ref[b], axis=0)
        fm2m_ref[b] = fm
        d = ((cm[0][:, None] - cm[0][None, :]) ** 2
             + (cm[1][:, None] - cm[1][None, :]) ** 2) \
            + (cm[2][:, None] - cm[2][None, :]) ** 2
        io = jax.lax.broadcasted_iota(jnp.int32, (m, m), 1)
        cols = []
        for j in range(KNN):
            dmin = jnp.min(d, axis=1, keepdims=True)
            idxj = jnp.min(jnp.where(d == dmin, io, m), axis=1)
            if j < KNN - 1:
                d = jnp.where(io == idxj[:, None], 1e30, d)
            ohj = (io == idxj[:, None]).astype(F32)
            nbc = _dt(cm, ohj)
            nbf = _dt(fm, ohj)
            cols.append(jnp.concatenate([nbc - cm, nbf], axis=0))
        in1 = jnp.concatenate(cols, axis=1)
        z1s.append(_mm(w1_ref[...], in1) + b1_ref[...])
    cntn = nb * KNN * m
    h1 = _norm_relu_multi(z1s, g1_ref, be1_ref, cntn)
    z2s = [_mm(w2_ref[...], h) + b2_ref[...] for h in h1]
    h2 = _norm_relu_multi(z2s, g2_ref, be2_ref, cntn)
    z3s = [_mm(w3_ref[...], h) + b3_ref[...] for h in h2]
    h3 = _norm_relu_multi(z3s, g3_ref, be3_ref, cntn)
    for b in range(nb):
        gm = h3[b][:, 0:m]
        for j in range(1, KNN):
            gm = jnp.maximum(gm, h3[b][:, j * m:(j + 1) * m])
        g_out_ref[b] = gm


def _tail_k(g_ref, fm2_ref, sums_ref, cnt_ref,
            ws1_ref, bs1_ref, gs1_ref, bes1_ref,
            ws2_ref, bs2_ref, gs2_ref, bes2_ref,
            wm1_ref, bm1_ref, gm1_ref, bem1_ref,
            wm2_ref, bm2_ref, gm2_ref, bem2_ref,
            wm3_ref, bm3_ref,
            cm_ref, kp_ref, sg_ref, *, nb, m, c2):
    cntn = nb * m
    z1s = [_mm(ws1_ref[...], g_ref[b]) + bs1_ref[...] for b in range(nb)]
    h1 = _norm_relu_multi(z1s, gs1_ref, bes1_ref, cntn)
    z2s = [_mm(ws2_ref[...], h) + bs2_ref[...] for h in h1]
    h2 = _norm_relu_multi(z2s, gs2_ref, bes2_ref, cntn)
    wm1 = wm1_ref[...]
    z3s = [_mm(wm1[:, :c2], fm2_ref[b]) + _mm(wm1[:, c2:], h2[b])
           + bm1_ref[...] for b in range(nb)]
    h3 = _norm_relu_multi(z3s, gm1_ref, bem1_ref, cntn)
    z4s = [_mm(wm2_ref[...], h) + bm2_ref[...] for h in h3]
    h4 = _norm_relu_multi(z4s, gm2_ref, bem2_ref, cntn)
    for b in range(nb):
        ks = _mm(wm3_ref[...], h4[b]) + bm3_ref[...]
        cm = sums_ref[b] / (cnt_ref[b] + 1e-5)
        cm_ref[b] = cm
        kp_ref[b] = ks[0:3, :] + cm
        s = ks[3, :]
        sg_ref[b] = jnp.maximum(s, 0.0) + jnp.log1p(jnp.exp(-jnp.abs(s))) \
            + 0.001


def _pb(p):
    return p["W"], p["b"].reshape(-1, 1), p["gamma"].reshape(-1, 1), \
        p["beta"].reshape(-1, 1)


def kernel(x, sn, node, params):
    nb, _, n = x.shape
    m = node.shape[2]
    pn = min(1024, n)
    t = n // pn
    cntN = nb * KS * n

    def full(shape):
        nd = len(shape)
        return pl.BlockSpec(shape, lambda b, i: (0,) * nd)

    def perb(shape):
        nd = len(shape) - 1
        return pl.BlockSpec(shape, lambda b, i: (b,) + (0,) * nd)

    def tile(shape, ax):
        def imap(b, i, _ax=ax):
            out = [0] * len(shape)
            out[0] = b
            out[_ax] = i
            return tuple(out)
        return pl.BlockSpec(shape, imap)

    idx, cnt, sums = pl.pallas_call(
        functools.partial(_assign_k, pn=pn, m=m),
        grid=(nb, t),
        in_specs=[tile((1, 3, pn), 2), perb((1, 3, m))],
        out_specs=[tile((1, KS, pn), 2), perb((1, 1, m)), perb((1, 3, m))],
        out_shape=[jax.ShapeDtypeStruct((nb, KS, n), jnp.int32),
                   jax.ShapeDtypeStruct((nb, 1, m), F32),
                   jax.ShapeDtypeStruct((nb, 3, m), F32)],
    )(x, node)

    w1, b1, g1, be1 = _pb(params["first_pn"][0])
    c = w1.shape[0]
    y, st = pl.pallas_call(
        functools.partial(_l1_k, pn=pn, m=m, cout=c),
        grid=(nb, t),
        in_specs=[tile((1, 3, pn), 2), tile((1, 3, pn), 2),
                  tile((1, KS, pn), 2), perb((1, 3, m)), perb((1, 1, m)),
                  full(w1.shape), full(b1.shape)],
        out_specs=[tile((1, c, KS, pn), 3), perb((1, 2, c))],
        out_shape=[jax.ShapeDtypeStruct((nb, c, KS, n), F32),
                   jax.ShapeDtypeStruct((nb, 2, c), F32)],
    )(x, sn, idx, sums, cnt, w1, b1)

    def mid_layer(y, st, p, gprev, beprev):
        w, b, g, be = _pb(p)
        cin = w.shape[1]
        cout = w.shape[0]
        return pl.pallas_call(
            functools.partial(_lmid_k, cnt=float(cntN), cout=cout),
            grid=(nb, t),
            in_specs=[tile((1, cin, KS, pn), 3), full((nb, 2, cin)),
                      full(gprev.shape), full(beprev.shape),
                      full(w.shape), full(b.shape)],
            out_specs=[tile((1, cout, KS, pn), 3), perb((1, 2, cout))],
            out_shape=[jax.ShapeDtypeStruct((nb, cout, KS, n), F32),
                       jax.ShapeDtypeStruct((nb, 2, cout), F32)],
        )(y, st, gprev, beprev, w, b), (g, be)

    _, _, g_c, be_c = _pb(params["first_pn"][0])
    (y, st), (g_c, be_c) = mid_layer(y, st, params["first_pn"][1], g_c, be_c)
    (y, st), (g_c, be_c) = mid_layer(y, st, params["first_pn"][2], g_c, be_c)

    nwb = 32 // nb
    idx_flat = idx.reshape(-1)
    f1 = pl.pallas_call(
        functools.partial(_normf_k, cnt=float(cntN)),
        grid=(nb, t),
        in_specs=[tile((1, c, KS, pn), 3), full((nb, 2, c)),
                  full(g_c.shape), full(be_c.shape)],
        out_specs=tile((1, c, KS, pn), 3),
        out_shape=jax.ShapeDtypeStruct((nb, c, KS, n), F32),
    )(y, st, g_c, be_c)
    f1m = _sc_segmax(f1, idx_flat, nb, c, n, m)

    w4, b4, g4, be4 = _pb(params["second_pn"][0])
    c2 = w4.shape[0]
    y, st = pl.pallas_call(
        functools.partial(_l4_k, pn=pn, m=m, cin=c, cout=c2),
        grid=(nb, t),
        in_specs=[tile((1, c, KS, pn), 3), tile((1, KS, pn), 2),
                  perb((1, nwb, c, m)), full(w4.shape), full(b4.shape)],
        out_specs=[tile((1, c2, KS, pn), 3), perb((1, 2, c2))],
        out_shape=[jax.ShapeDtypeStruct((nb, c2, KS, n), F32),
                   jax.ShapeDtypeStruct((nb, 2, c2), F32)],
    )(f1, idx, f1m, w4, b4)

    (y, st), (g_c, be_c) = mid_layer(y, st, params["second_pn"][1], g4, be4)

    f2 = pl.pallas_call(
        functools.partial(_normf_k, cnt=float(cntN)),
        grid=(nb, t),
        in_specs=[tile((1, c2, KS, pn), 3), full((nb, 2, c2)),
                  full(g_c.shape), full(be_c.shape)],
        out_specs=tile((1, c2, KS, pn), 3),
        out_shape=jax.ShapeDtypeStruct((nb, c2, KS, n), F32),
    )(y, st, g_c, be_c)
    f2p = _sc_segmax(f2, idx_flat, nb, c2, n, m)

    kf = [_pb(p) for p in params["knn_first"]]
    ck = kf[0][0].shape[0]
    g_nodes, f2m = pl.pallas_call(
        functools.partial(_knn_k, nb=nb, m=m),
        out_shape=[jax.ShapeDtypeStruct((nb, ck, m), F32),
                   jax.ShapeDtypeStruct((nb, c2, m), F32)],
    )(sums, cnt, f2p, *kf[0], *kf[1], *kf[2])

    ksnd = [_pb(p) for p in params["knn_second"]]
    m1 = _pb(params["mlp1"][0])
    m2 = _pb(params["mlp2"][0])
    w3p = params["mlp3"][0]["W"]
    b3p = params["mlp3"][0]["b"].reshape(-1, 1)
    cm_out, kp, sg = pl.pallas_call(
        functools.partial(_tail_k, nb=nb, m=m, c2=c2),
        out_shape=[jax.ShapeDtypeStruct((nb, 3, m), F32),
                   jax.ShapeDtypeStruct((nb, 3, m), F32),
                   jax.ShapeDtypeStruct((nb, m), F32)],
    )(g_nodes, f2m, sums, cnt, *ksnd[0], *ksnd[1], *m1, *m2, w3p, b3p)

    return (cm_out, kp, sg)


# speedup vs baseline: 100.6350x; 1.7156x over previous
import functools

import jax
import jax.numpy as jnp
from jax import lax
from jax.experimental import pallas as pl
from jax.experimental.pallas import tpu as pltpu
from jax.experimental.pallas import tpu_sc as plsc

F32 = jnp.float32
KS = 3
KNN = 16
EPS = 1e-5
SEG = 128


def _oh(idx_vec, n, m):
    io = jax.lax.broadcasted_iota(jnp.int32, (n, m), 1)
    return (io == idx_vec[:, None]).astype(F32)


def _dt(a, b):
    return jax.lax.dot_general(a, b, (((1,), (1,)), ((), ())),
                               preferred_element_type=F32)


def _mm(a, b):
    return jnp.dot(a, b, preferred_element_type=F32)


def _assign_k(x_ref, node_ref, idx_ref, cnt_ref, sums_ref, *, pn, m):
    t = pl.program_id(1)

    @pl.when(t == 0)
    def _():
        cnt_ref[...] = jnp.zeros_like(cnt_ref)
        sums_ref[...] = jnp.zeros_like(sums_ref)

    x = x_ref[0]
    nd = node_ref[0]
    d = ((x[0][:, None] - nd[0][None, :]) ** 2
         + (x[1][:, None] - nd[1][None, :]) ** 2) \
        + (x[2][:, None] - nd[2][None, :]) ** 2
    io = jax.lax.broadcasted_iota(jnp.int32, (pn, m), 1)
    cnt = jnp.zeros((1, m), F32)
    sm = jnp.zeros((3, m), F32)
    for j in range(KS):
        dmin = jnp.min(d, axis=1, keepdims=True)
        idxj = jnp.min(jnp.where(d == dmin, io, m), axis=1)
        idx_ref[0, j, :] = idxj
        ohj = (io == idxj[:, None]).astype(F32)
        cnt = cnt + jnp.sum(ohj, axis=0, keepdims=True)
        sm = sm + jax.lax.dot_general(x, ohj, (((1,), (0,)), ((), ())),
                                      preferred_element_type=F32)
        if j < KS - 1:
            d = jnp.where(io == idxj[:, None], 1e30, d)
    cnt_ref[0] += cnt
    sums_ref[0] += sm


def _scale_shift(st, g_ref, be_ref, cnt):
    tot1 = jnp.sum(st[:, 0, :], axis=0)
    tot2 = jnp.sum(st[:, 1, :], axis=0)
    mean = tot1 / cnt
    var = tot2 / cnt - mean * mean
    scale = g_ref[...][:, 0] / jnp.sqrt(var + EPS)
    shift = be_ref[...][:, 0] - mean * scale
    return scale[:, None], shift[:, None]


def _l1_k(x_ref, sn_ref, idx_ref, sums_ref, cnt_ref, w_ref, b_ref,
          y_ref, st_ref, *, pn, m, cout):
    t = pl.program_id(1)

    @pl.when(t == 0)
    def _():
        st_ref[...] = jnp.zeros_like(st_ref)

    cm = sums_ref[0] / (cnt_ref[0] + 1e-5)
    W = w_ref[...]
    bb = b_ref[...]
    x = x_ref[0]
    sn = sn_ref[0]
    s1 = jnp.zeros((cout,), F32)
    s2 = jnp.zeros((cout,), F32)
    for k in range(KS):
        oh = _oh(idx_ref[0, k, :], pn, m)
        cen = _dt(cm, oh)
        xa = jnp.concatenate([x - cen, sn], axis=0)
        y = _mm(W, xa) + bb
        y_ref[0, :, k, :] = y
        s1 += jnp.sum(y, axis=1)
        s2 += jnp.sum(y * y, axis=1)
    st_ref[0, 0, :] += s1
    st_ref[0, 1, :] += s2


def _lmid_k(y_ref, st_ref, g_ref, be_ref, w_ref, b_ref, o_ref, so_ref,
            *, cnt, cout):
    t = pl.program_id(1)

    @pl.when(t == 0)
    def _():
        so_ref[...] = jnp.zeros_like(so_ref)

    scale, shift = _scale_shift(st_ref[...], g_ref, be_ref, cnt)
    W = w_ref[...]
    bb = b_ref[...]
    s1 = jnp.zeros((cout,), F32)
    s2 = jnp.zeros((cout,), F32)
    for k in range(KS):
        h = jnp.maximum(y_ref[0, :, k, :] * scale + shift, 0.0)
        z = _mm(W, h) + bb
        o_ref[0, :, k, :] = z
        s1 += jnp.sum(z, axis=1)
        s2 += jnp.sum(z * z, axis=1)
    so_ref[0, 0, :] += s1
    so_ref[0, 1, :] += s2


def _l4_k(f_ref, idx_ref, fm_ref, w_ref, b_ref, o_ref, so_ref,
          *, pn, m, cin, cout):
    t = pl.program_id(1)

    @pl.when(t == 0)
    def _():
        so_ref[...] = jnp.zeros_like(so_ref)

    W = w_ref[...]
    Wf = W[:, :cin]
    Ws = W[:, cin:]
    bb = b_ref[...]
    fmT = jnp.max(fm_ref[0], axis=0)
    s1 = jnp.zeros((cout,), F32)
    s2 = jnp.zeros((cout,), F32)
    for k in range(KS):
        fT = f_ref[0, k, :, :]
        oh = _oh(idx_ref[0, k, :], pn, m)
        scatT = _mm(oh, fmT)
        z = _dt(Wf, fT) + _dt(Ws, scatT) + bb
        o_ref[0, :, k, :] = z
        s1 += jnp.sum(z, axis=1)
        s2 += jnp.sum(z * z, axis=1)
    so_ref[0, 0, :] += s1
    so_ref[0, 1, :] += s2


def _normf_k(y_ref, st_ref, g_ref, be_ref, f_ref, *, cnt):
    scale, shift = _scale_shift(st_ref[...], g_ref, be_ref, cnt)
    for k in range(KS):
        f = jnp.maximum(y_ref[0, :, k, :] * scale + shift, 0.0)
        f_ref[0, k, :, :] = f.T


def _dg(v, i):
    return lax.gather(
        v, i[:, None],
        lax.GatherDimensionNumbers(offset_dims=(), collapsed_slice_dims=(0,),
                                   start_index_map=(0,)),
        (1,), mode=lax.GatherScatterMode.PROMISE_IN_BOUNDS)


def _sc_segmax(ft_flat, idx_flat, nb, c, n, m):
    nwb = 32 // nb
    ch = n // nwb
    mesh = plsc.VectorSubcoreMesh(core_axis_name="c", subcore_axis_name="s")

    @functools.partial(
        pl.kernel, mesh=mesh,
        compiler_params=pltpu.CompilerParams(needs_layout_passes=False),
        out_type=jax.ShapeDtypeStruct((nb * nwb * m * c,), F32),
        scratch_types=[
            pltpu.VMEM((ch * c,), F32),
            pltpu.VMEM((ch,), jnp.int32),
            pltpu.VMEM((m * c,), F32),
        ],
    )
    def k(ft_hbm, idx_hbm, out_hbm, fv, iv, acc):
        ci = lax.axis_index("c")
        si = lax.axis_index("s")
        w = si * 2 + ci
        b = w // nwb
        ww = w % nwb
        zero = jnp.zeros((16,), F32)

        def zbody(i, _):
            acc[pl.ds(i * 16, 16)] = zero
            return 0

        lax.fori_loop(0, m * c // 16, zbody, 0)

        io = lax.iota(jnp.int32, 16)
        for kk in range(KS):
            pltpu.sync_copy(
                idx_hbm.at[pl.ds((b * KS + kk) * n + ww * ch, ch)], iv)
            pltpu.sync_copy(
                ft_hbm.at[pl.ds(((b * KS + kk) * n + ww * ch) * c, ch * c)],
                fv)

            def gbody(g, _):
                ivv = iv[pl.ds(g * 16, 16)]
                for l in range(16):
                    mvec = _dg(ivv, jnp.full((16,), l, jnp.int32))
                    base = mvec * c + io
                    pc = g * 16 + l
                    for j in range(c // 16):
                        vv = fv[pl.ds(pc * c + j * 16, 16)]
                        idxv = base + j * 16
                        cur = plsc.load_gather(acc, [idxv])
                        plsc.store_scatter(acc, [idxv],
                                           jnp.maximum(cur, vv))
                return 0

            lax.fori_loop(0, ch // 16, gbody, 0)
        pltpu.sync_copy(acc, out_hbm.at[pl.ds((b * nwb + ww) * m * c,
                                              m * c)])

    return k(ft_flat, idx_flat).reshape(nb, nwb, m, c)


def _norm_relu_multi(zs, g_ref, be_ref, cnt):
    s1 = zs[0].sum(axis=1)
    for z in zs[1:]:
        s1 = s1 + z.sum(axis=1)
    mean = s1 / cnt
    s2 = ((zs[0] - mean[:, None]) ** 2).sum(axis=1)
    for z in zs[1:]:
        s2 = s2 + ((z - mean[:, None]) ** 2).sum(axis=1)
    var = s2 / cnt
    scale = (g_ref[...][:, 0] / jnp.sqrt(var + EPS))[:, None]
    shift = (be_ref[...][:, 0])[:, None] - mean[:, None] * scale
    return [jnp.maximum(z * scale + shift, 0.0) for z in zs]


def _knn_k(sums_ref, cnt_ref, fm2_ref,
           w1_ref, b1_ref, g1_ref, be1_ref,
           w2_ref, b2_ref, g2_ref, be2_ref,
           w3_ref, b3_ref, g3_ref, be3_ref,
           g_out_ref, fm2m_ref, *, nb, m):
    z1s = []
    for b in range(nb):
        cm = sums_ref[b] / (cnt_ref[b] + 1e-5)
        fm = jnp.max(fm2_ref[b], axis=0).T
        fm2m_ref[b] = fm
        d = ((cm[0][:, None] - cm[0][None, :]) ** 2
             + (cm[1][:, None] - cm[1][None, :]) ** 2) \
            + (cm[2][:, None] - cm[2][None, :]) ** 2
        io = jax.lax.broadcasted_iota(jnp.int32, (m, m), 1)
        cols = []
        for j in range(KNN):
            dmin = jnp.min(d, axis=1, keepdims=True)
            idxj = jnp.min(jnp.where(d == dmin, io, m), axis=1)
            if j < KNN - 1:
                d = jnp.where(io == idxj[:, None], 1e30, d)
            ohj = (io == idxj[:, None]).astype(F32)
            nbc = _dt(cm, ohj)
            nbf = _dt(fm, ohj)
            cols.append(jnp.concatenate([nbc - cm, nbf], axis=0))
        in1 = jnp.concatenate(cols, axis=1)
        z1s.append(_mm(w1_ref[...], in1) + b1_ref[...])
    cntn = nb * KNN * m
    h1 = _norm_relu_multi(z1s, g1_ref, be1_ref, cntn)
    z2s = [_mm(w2_ref[...], h) + b2_ref[...] for h in h1]
    h2 = _norm_relu_multi(z2s, g2_ref, be2_ref, cntn)
    z3s = [_mm(w3_ref[...], h) + b3_ref[...] for h in h2]
    h3 = _norm_relu_multi(z3s, g3_ref, be3_ref, cntn)
    for b in range(nb):
        gm = h3[b][:, 0:m]
        for j in range(1, KNN):
            gm = jnp.maximum(gm, h3[b][:, j * m:(j + 1) * m])
        g_out_ref[b] = gm


def _tail_k(g_ref, fm2_ref, sums_ref, cnt_ref,
            ws1_ref, bs1_ref, gs1_ref, bes1_ref,
            ws2_ref, bs2_ref, gs2_ref, bes2_ref,
            wm1_ref, bm1_ref, gm1_ref, bem1_ref,
            wm2_ref, bm2_ref, gm2_ref, bem2_ref,
            wm3_ref, bm3_ref,
            cm_ref, kp_ref, sg_ref, *, nb, m, c2):
    cntn = nb * m
    z1s = [_mm(ws1_ref[...], g_ref[b]) + bs1_ref[...] for b in range(nb)]
    h1 = _norm_relu_multi(z1s, gs1_ref, bes1_ref, cntn)
    z2s = [_mm(ws2_ref[...], h) + bs2_ref[...] for h in h1]
    h2 = _norm_relu_multi(z2s, gs2_ref, bes2_ref, cntn)
    wm1 = wm1_ref[...]
    z3s = [_mm(wm1[:, :c2], fm2_ref[b]) + _mm(wm1[:, c2:], h2[b])
           + bm1_ref[...] for b in range(nb)]
    h3 = _norm_relu_multi(z3s, gm1_ref, bem1_ref, cntn)
    z4s = [_mm(wm2_ref[...], h) + bm2_ref[...] for h in h3]
    h4 = _norm_relu_multi(z4s, gm2_ref, bem2_ref, cntn)
    for b in range(nb):
        ks = _mm(wm3_ref[...], h4[b]) + bm3_ref[...]
        cm = sums_ref[b] / (cnt_ref[b] + 1e-5)
        cm_ref[b] = cm
        kp_ref[b] = ks[0:3, :] + cm
        s = ks[3, :]
        sg_ref[b] = jnp.maximum(s, 0.0) + jnp.log1p(jnp.exp(-jnp.abs(s))) \
            + 0.001


def _pb(p):
    return p["W"], p["b"].reshape(-1, 1), p["gamma"].reshape(-1, 1), \
        p["beta"].reshape(-1, 1)


def kernel(x, sn, node, params):
    nb, _, n = x.shape
    m = node.shape[2]
    pn = min(1024, n)
    t = n // pn
    cntN = nb * KS * n

    def full(shape):
        nd = len(shape)
        return pl.BlockSpec(shape, lambda b, i: (0,) * nd)

    def perb(shape):
        nd = len(shape) - 1
        return pl.BlockSpec(shape, lambda b, i: (b,) + (0,) * nd)

    def tile(shape, ax):
        def imap(b, i, _ax=ax):
            out = [0] * len(shape)
            out[0] = b
            out[_ax] = i
            return tuple(out)
        return pl.BlockSpec(shape, imap)

    idx, cnt, sums = pl.pallas_call(
        functools.partial(_assign_k, pn=pn, m=m),
        grid=(nb, t),
        in_specs=[tile((1, 3, pn), 2), perb((1, 3, m))],
        out_specs=[tile((1, KS, pn), 2), perb((1, 1, m)), perb((1, 3, m))],
        out_shape=[jax.ShapeDtypeStruct((nb, KS, n), jnp.int32),
                   jax.ShapeDtypeStruct((nb, 1, m), F32),
                   jax.ShapeDtypeStruct((nb, 3, m), F32)],
    )(x, node)

    w1, b1, g1, be1 = _pb(params["first_pn"][0])
    c = w1.shape[0]
    y, st = pl.pallas_call(
        functools.partial(_l1_k, pn=pn, m=m, cout=c),
        grid=(nb, t),
        in_specs=[tile((1, 3, pn), 2), tile((1, 3, pn), 2),
                  tile((1, KS, pn), 2), perb((1, 3, m)), perb((1, 1, m)),
                  full(w1.shape), full(b1.shape)],
        out_specs=[tile((1, c, KS, pn), 3), perb((1, 2, c))],
        out_shape=[jax.ShapeDtypeStruct((nb, c, KS, n), F32),
                   jax.ShapeDtypeStruct((nb, 2, c), F32)],
    )(x, sn, idx, sums, cnt, w1, b1)

    def mid_layer(y, st, p, gprev, beprev):
        w, b, g, be = _pb(p)
        cin = w.shape[1]
        cout = w.shape[0]
        return pl.pallas_call(
            functools.partial(_lmid_k, cnt=float(cntN), cout=cout),
            grid=(nb, t),
            in_specs=[tile((1, cin, KS, pn), 3), full((nb, 2, cin)),
                      full(gprev.shape), full(beprev.shape),
                      full(w.shape), full(b.shape)],
            out_specs=[tile((1, cout, KS, pn), 3), perb((1, 2, cout))],
            out_shape=[jax.ShapeDtypeStruct((nb, cout, KS, n), F32),
                       jax.ShapeDtypeStruct((nb, 2, cout), F32)],
        )(y, st, gprev, beprev, w, b), (g, be)

    _, _, g_c, be_c = _pb(params["first_pn"][0])
    (y, st), (g_c, be_c) = mid_layer(y, st, params["first_pn"][1], g_c, be_c)
    (y, st), (g_c, be_c) = mid_layer(y, st, params["first_pn"][2], g_c, be_c)

    nwb = 32 // nb
    idx_flat = idx.reshape(-1)
    f1 = pl.pallas_call(
        functools.partial(_normf_k, cnt=float(cntN)),
        grid=(nb, t),
        in_specs=[tile((1, c, KS, pn), 3), full((nb, 2, c)),
                  full(g_c.shape), full(be_c.shape)],
        out_specs=tile((1, KS, pn, c), 2),
        out_shape=jax.ShapeDtypeStruct((nb, KS, n, c), F32),
    )(y, st, g_c, be_c)
    f1m = _sc_segmax(f1.reshape(-1), idx_flat, nb, c, n, m)

    w4, b4, g4, be4 = _pb(params["second_pn"][0])
    c2 = w4.shape[0]
    y, st = pl.pallas_call(
        functools.partial(_l4_k, pn=pn, m=m, cin=c, cout=c2),
        grid=(nb, t),
        in_specs=[tile((1, KS, pn, c), 2), tile((1, KS, pn), 2),
                  perb((1, nwb, m, c)), full(w4.shape), full(b4.shape)],
        out_specs=[tile((1, c2, KS, pn), 3), perb((1, 2, c2))],
        out_shape=[jax.ShapeDtypeStruct((nb, c2, KS, n), F32),
                   jax.ShapeDtypeStruct((nb, 2, c2), F32)],
    )(f1, idx, f1m, w4, b4)

    (y, st), (g_c, be_c) = mid_layer(y, st, params["second_pn"][1], g4, be4)

    f2 = pl.pallas_call(
        functools.partial(_normf_k, cnt=float(cntN)),
        grid=(nb, t),
        in_specs=[tile((1, c2, KS, pn), 3), full((nb, 2, c2)),
                  full(g_c.shape), full(be_c.shape)],
        out_specs=tile((1, KS, pn, c2), 2),
        out_shape=jax.ShapeDtypeStruct((nb, KS, n, c2), F32),
    )(y, st, g_c, be_c)
    f2p = _sc_segmax(f2.reshape(-1), idx_flat, nb, c2, n, m)

    kf = [_pb(p) for p in params["knn_first"]]
    ck = kf[0][0].shape[0]
    g_nodes, f2m = pl.pallas_call(
        functools.partial(_knn_k, nb=nb, m=m),
        out_shape=[jax.ShapeDtypeStruct((nb, ck, m), F32),
                   jax.ShapeDtypeStruct((nb, c2, m), F32)],
    )(sums, cnt, f2p, *kf[0], *kf[1], *kf[2])

    ksnd = [_pb(p) for p in params["knn_second"]]
    m1 = _pb(params["mlp1"][0])
    m2 = _pb(params["mlp2"][0])
    w3p = params["mlp3"][0]["W"]
    b3p = params["mlp3"][0]["b"].reshape(-1, 1)
    cm_out, kp, sg = pl.pallas_call(
        functools.partial(_tail_k, nb=nb, m=m, c2=c2),
        out_shape=[jax.ShapeDtypeStruct((nb, 3, m), F32),
                   jax.ShapeDtypeStruct((nb, 3, m), F32),
                   jax.ShapeDtypeStruct((nb, m), F32)],
    )(g_nodes, f2m, sums, cnt, *ksnd[0], *ksnd[1], *m1, *m2, w3p, b3p)

    return (cm_out, kp, sg)


# speedup vs baseline: 107.9510x; 1.0727x over previous
import functools

import jax
import jax.numpy as jnp
from jax import lax
from jax.experimental import pallas as pl
from jax.experimental.pallas import tpu as pltpu
from jax.experimental.pallas import tpu_sc as plsc

F32 = jnp.float32
KS = 3
KNN = 16
EPS = 1e-5
SEG = 128


def _oh(idx_vec, n, m):
    io = jax.lax.broadcasted_iota(jnp.int32, (n, m), 1)
    return (io == idx_vec[:, None]).astype(F32)


def _dt(a, b):
    return jax.lax.dot_general(a, b, (((1,), (1,)), ((), ())),
                               preferred_element_type=F32)


def _mm(a, b):
    return jnp.dot(a, b, preferred_element_type=F32)


def _assign_k(x_ref, node_ref, idx_ref, cnt_ref, sums_ref, *, pn, m):
    t = pl.program_id(1)

    @pl.when(t == 0)
    def _():
        cnt_ref[...] = jnp.zeros_like(cnt_ref)
        sums_ref[...] = jnp.zeros_like(sums_ref)

    x = x_ref[0]
    nd = node_ref[0]
    d = ((x[0][:, None] - nd[0][None, :]) ** 2
         + (x[1][:, None] - nd[1][None, :]) ** 2) \
        + (x[2][:, None] - nd[2][None, :]) ** 2
    io = jax.lax.broadcasted_iota(jnp.int32, (pn, m), 1)
    cnt = jnp.zeros((1, m), F32)
    sm = jnp.zeros((3, m), F32)
    for j in range(KS):
        dmin = jnp.min(d, axis=1, keepdims=True)
        idxj = jnp.min(jnp.where(d == dmin, io, m), axis=1)
        idx_ref[0, j, :] = idxj
        ohj = (io == idxj[:, None]).astype(F32)
        cnt = cnt + jnp.sum(ohj, axis=0, keepdims=True)
        sm = sm + jax.lax.dot_general(x, ohj, (((1,), (0,)), ((), ())),
                                      preferred_element_type=F32)
        if j < KS - 1:
            d = jnp.where(io == idxj[:, None], 1e30, d)
    cnt_ref[0] += cnt
    sums_ref[0] += sm


def _scale_shift(st, g_ref, be_ref, cnt):
    tot1 = jnp.sum(st[:, 0, :], axis=0)
    tot2 = jnp.sum(st[:, 1, :], axis=0)
    mean = tot1 / cnt
    var = tot2 / cnt - mean * mean
    scale = g_ref[...][:, 0] / jnp.sqrt(var + EPS)
    shift = be_ref[...][:, 0] - mean * scale
    return scale[:, None], shift[:, None]


def _l1_k(x_ref, sn_ref, idx_ref, sums_ref, cnt_ref, w_ref, b_ref,
          y_ref, st_ref, *, pn, m, cout):
    t = pl.program_id(1)

    @pl.when(t == 0)
    def _():
        st_ref[...] = jnp.zeros_like(st_ref)

    cm = sums_ref[0] / (cnt_ref[0] + 1e-5)
    W = w_ref[...]
    bb = b_ref[...]
    x = x_ref[0]
    sn = sn_ref[0]
    s1 = jnp.zeros((cout,), F32)
    s2 = jnp.zeros((cout,), F32)
    for k in range(KS):
        oh = _oh(idx_ref[0, k, :], pn, m)
        cen = _dt(cm, oh)
        xa = jnp.concatenate([x - cen, sn], axis=0)
        y = _mm(W, xa) + bb
        y_ref[0, :, k, :] = y
        s1 += jnp.sum(y, axis=1)
        s2 += jnp.sum(y * y, axis=1)
    st_ref[0, 0, :] += s1
    st_ref[0, 1, :] += s2


def _lmid_k(y_ref, st_ref, g_ref, be_ref, w_ref, b_ref, o_ref, so_ref,
            *, cnt, cout):
    t = pl.program_id(1)

    @pl.when(t == 0)
    def _():
        so_ref[...] = jnp.zeros_like(so_ref)

    scale, shift = _scale_shift(st_ref[...], g_ref, be_ref, cnt)
    W = w_ref[...]
    bb = b_ref[...]
    s1 = jnp.zeros((cout,), F32)
    s2 = jnp.zeros((cout,), F32)
    for k in range(KS):
        h = jnp.maximum(y_ref[0, :, k, :] * scale + shift, 0.0)
        z = _mm(W, h) + bb
        o_ref[0, :, k, :] = z
        s1 += jnp.sum(z, axis=1)
        s2 += jnp.sum(z * z, axis=1)
    so_ref[0, 0, :] += s1
    so_ref[0, 1, :] += s2


def _l4_k(f_ref, idx_ref, fm_ref, w_ref, b_ref, o_ref, so_ref,
          *, pn, m, cin, cout):
    t = pl.program_id(1)

    @pl.when(t == 0)
    def _():
        so_ref[...] = jnp.zeros_like(so_ref)

    W = w_ref[...]
    Wf = W[:, :cin]
    Ws = W[:, cin:]
    bb = b_ref[...]
    fmT = jnp.max(fm_ref[0], axis=0)
    s1 = jnp.zeros((cout,), F32)
    s2 = jnp.zeros((cout,), F32)
    for k in range(KS):
        fT = f_ref[0, k, :, :]
        oh = _oh(idx_ref[0, k, :], pn, m)
        scatT = _mm(oh, fmT)
        z = _dt(Wf, fT) + _dt(Ws, scatT) + bb
        o_ref[0, :, k, :] = z
        s1 += jnp.sum(z, axis=1)
        s2 += jnp.sum(z * z, axis=1)
    so_ref[0, 0, :] += s1
    so_ref[0, 1, :] += s2


def _normf_k(y_ref, st_ref, g_ref, be_ref, f_ref, *, cnt):
    scale, shift = _scale_shift(st_ref[...], g_ref, be_ref, cnt)
    for k in range(KS):
        f = jnp.maximum(y_ref[0, :, k, :] * scale + shift, 0.0)
        f_ref[0, k, :, :] = f.T


def _dg(v, i):
    return lax.gather(
        v, i[:, None],
        lax.GatherDimensionNumbers(offset_dims=(), collapsed_slice_dims=(0,),
                                   start_index_map=(0,)),
        (1,), mode=lax.GatherScatterMode.PROMISE_IN_BOUNDS)


def _sc_segmax(ft_flat, idx_flat, nb, c, n, m):
    nwb = 32 // nb
    ch = n // nwb
    mesh = plsc.VectorSubcoreMesh(core_axis_name="c", subcore_axis_name="s")

    @functools.partial(
        pl.kernel, mesh=mesh,
        compiler_params=pltpu.CompilerParams(needs_layout_passes=False),
        out_type=jax.ShapeDtypeStruct((nb * nwb * m * c,), F32),
        scratch_types=[
            pltpu.VMEM((ch * c,), F32),
            pltpu.VMEM((ch,), jnp.int32),
            pltpu.VMEM((m * c,), F32),
        ],
    )
    def k(ft_hbm, idx_hbm, out_hbm, fv, iv, acc):
        ci = lax.axis_index("c")
        si = lax.axis_index("s")
        w = si * 2 + ci
        b = w // nwb
        ww = w % nwb
        zero = jnp.zeros((16,), F32)

        def zbody(i, _):
            acc[pl.ds(i * 16, 16)] = zero
            return 0

        lax.fori_loop(0, m * c // 16, zbody, 0)

        io = lax.iota(jnp.int32, 16)
        for kk in range(KS):
            pltpu.sync_copy(
                idx_hbm.at[pl.ds((b * KS + kk) * n + ww * ch, ch)], iv)
            pltpu.sync_copy(
                ft_hbm.at[pl.ds(((b * KS + kk) * n + ww * ch) * c, ch * c)],
                fv)

            def gbody(g, _):
                ivv = iv[pl.ds(g * 16, 16)]
                for l in range(16):
                    mvec = _dg(ivv, jnp.full((16,), l, jnp.int32))
                    base = mvec * c + io
                    pc = g * 16 + l
                    for j in range(c // 16):
                        vv = fv[pl.ds(pc * c + j * 16, 16)]
                        idxv = base + j * 16
                        cur = plsc.load_gather(acc, [idxv])
                        plsc.store_scatter(acc, [idxv],
                                           jnp.maximum(cur, vv))
                return 0

            lax.fori_loop(0, ch // 16, gbody, 0)
        pltpu.sync_copy(acc, out_hbm.at[pl.ds((b * nwb + ww) * m * c,
                                              m * c)])

    return k(ft_flat, idx_flat).reshape(nb, nwb, m, c)


def _norm_relu_multi(zs, g_ref, be_ref, cnt):
    s1 = zs[0].sum(axis=1)
    for z in zs[1:]:
        s1 = s1 + z.sum(axis=1)
    mean = s1 / cnt
    s2 = ((zs[0] - mean[:, None]) ** 2).sum(axis=1)
    for z in zs[1:]:
        s2 = s2 + ((z - mean[:, None]) ** 2).sum(axis=1)
    var = s2 / cnt
    scale = (g_ref[...][:, 0] / jnp.sqrt(var + EPS))[:, None]
    shift = (be_ref[...][:, 0])[:, None] - mean[:, None] * scale
    return [jnp.maximum(z * scale + shift, 0.0) for z in zs]


def _knn_k(sums_ref, cnt_ref, fm2_ref,
           w1_ref, b1_ref, g1_ref, be1_ref,
           w2_ref, b2_ref, g2_ref, be2_ref,
           w3_ref, b3_ref, g3_ref, be3_ref,
           g_out_ref, fm2m_ref, *, nb, m):
    z1s = []
    for b in range(nb):
        cm = sums_ref[b] / (cnt_ref[b] + 1e-5)
        fm = jnp.max(fm2_ref[b], axis=0).T
        fm2m_ref[b] = fm
        d = ((cm[0][:, None] - cm[0][None, :]) ** 2
             + (cm[1][:, None] - cm[1][None, :]) ** 2) \
            + (cm[2][:, None] - cm[2][None, :]) ** 2
        io = jax.lax.broadcasted_iota(jnp.int32, (m, m), 1)
        cols = []
        for j in range(KNN):
            dmin = jnp.min(d, axis=1, keepdims=True)
            idxj = jnp.min(jnp.where(d == dmin, io, m), axis=1)
            if j < KNN - 1:
                d = jnp.where(io == idxj[:, None], 1e30, d)
            ohj = (io == idxj[:, None]).astype(F32)
            nbc = _dt(cm, ohj)
            nbf = _dt(fm, ohj)
            cols.append(jnp.concatenate([nbc - cm, nbf], axis=0))
        in1 = jnp.concatenate(cols, axis=1)
        z1s.append(_mm(w1_ref[...], in1) + b1_ref[...])
    cntn = nb * KNN * m
    h1 = _norm_relu_multi(z1s, g1_ref, be1_ref, cntn)
    z2s = [_mm(w2_ref[...], h) + b2_ref[...] for h in h1]
    h2 = _norm_relu_multi(z2s, g2_ref, be2_ref, cntn)
    z3s = [_mm(w3_ref[...], h) + b3_ref[...] for h in h2]
    h3 = _norm_relu_multi(z3s, g3_ref, be3_ref, cntn)
    for b in range(nb):
        gm = h3[b][:, 0:m]
        for j in range(1, KNN):
            gm = jnp.maximum(gm, h3[b][:, j * m:(j + 1) * m])
        g_out_ref[b] = gm


def _tail_k(g_ref, fm2_ref, sums_ref, cnt_ref,
            ws1_ref, bs1_ref, gs1_ref, bes1_ref,
            ws2_ref, bs2_ref, gs2_ref, bes2_ref,
            wm1_ref, bm1_ref, gm1_ref, bem1_ref,
            wm2_ref, bm2_ref, gm2_ref, bem2_ref,
            wm3_ref, bm3_ref,
            cm_ref, kp_ref, sg_ref, *, nb, m, c2):
    cntn = nb * m
    z1s = [_mm(ws1_ref[...], g_ref[b]) + bs1_ref[...] for b in range(nb)]
    h1 = _norm_relu_multi(z1s, gs1_ref, bes1_ref, cntn)
    z2s = [_mm(ws2_ref[...], h) + bs2_ref[...] for h in h1]
    h2 = _norm_relu_multi(z2s, gs2_ref, bes2_ref, cntn)
    wm1 = wm1_ref[...]
    z3s = [_mm(wm1[:, :c2], fm2_ref[b]) + _mm(wm1[:, c2:], h2[b])
           + bm1_ref[...] for b in range(nb)]
    h3 = _norm_relu_multi(z3s, gm1_ref, bem1_ref, cntn)
    z4s = [_mm(wm2_ref[...], h) + bm2_ref[...] for h in h3]
    h4 = _norm_relu_multi(z4s, gm2_ref, bem2_ref, cntn)
    for b in range(nb):
        ks = _mm(wm3_ref[...], h4[b]) + bm3_ref[...]
        cm = sums_ref[b] / (cnt_ref[b] + 1e-5)
        cm_ref[b] = cm
        kp_ref[b] = ks[0:3, :] + cm
        s = ks[3, :]
        sg_ref[b] = jnp.maximum(s, 0.0) + jnp.log1p(jnp.exp(-jnp.abs(s))) \
            + 0.001


def _pb(p):
    return p["W"], p["b"].reshape(-1, 1), p["gamma"].reshape(-1, 1), \
        p["beta"].reshape(-1, 1)


def kernel(x, sn, node, params):
    nb, _, n = x.shape
    m = node.shape[2]
    pn = min(2048, n)
    t = n // pn
    cntN = nb * KS * n

    def full(shape):
        nd = len(shape)
        return pl.BlockSpec(shape, lambda b, i: (0,) * nd)

    def perb(shape):
        nd = len(shape) - 1
        return pl.BlockSpec(shape, lambda b, i: (b,) + (0,) * nd)

    def tile(shape, ax):
        def imap(b, i, _ax=ax):
            out = [0] * len(shape)
            out[0] = b
            out[_ax] = i
            return tuple(out)
        return pl.BlockSpec(shape, imap)

    idx, cnt, sums = pl.pallas_call(
        functools.partial(_assign_k, pn=pn, m=m),
        grid=(nb, t),
        in_specs=[tile((1, 3, pn), 2), perb((1, 3, m))],
        out_specs=[tile((1, KS, pn), 2), perb((1, 1, m)), perb((1, 3, m))],
        out_shape=[jax.ShapeDtypeStruct((nb, KS, n), jnp.int32),
                   jax.ShapeDtypeStruct((nb, 1, m), F32),
                   jax.ShapeDtypeStruct((nb, 3, m), F32)],
    )(x, node)

    w1, b1, g1, be1 = _pb(params["first_pn"][0])
    c = w1.shape[0]
    y, st = pl.pallas_call(
        functools.partial(_l1_k, pn=pn, m=m, cout=c),
        grid=(nb, t),
        in_specs=[tile((1, 3, pn), 2), tile((1, 3, pn), 2),
                  tile((1, KS, pn), 2), perb((1, 3, m)), perb((1, 1, m)),
                  full(w1.shape), full(b1.shape)],
        out_specs=[tile((1, c, KS, pn), 3), perb((1, 2, c))],
        out_shape=[jax.ShapeDtypeStruct((nb, c, KS, n), F32),
                   jax.ShapeDtypeStruct((nb, 2, c), F32)],
    )(x, sn, idx, sums, cnt, w1, b1)

    def mid_layer(y, st, p, gprev, beprev):
        w, b, g, be = _pb(p)
        cin = w.shape[1]
        cout = w.shape[0]
        return pl.pallas_call(
            functools.partial(_lmid_k, cnt=float(cntN), cout=cout),
            grid=(nb, t),
            in_specs=[tile((1, cin, KS, pn), 3), full((nb, 2, cin)),
                      full(gprev.shape), full(beprev.shape),
                      full(w.shape), full(b.shape)],
            out_specs=[tile((1, cout, KS, pn), 3), perb((1, 2, cout))],
            out_shape=[jax.ShapeDtypeStruct((nb, cout, KS, n), F32),
                       jax.ShapeDtypeStruct((nb, 2, cout), F32)],
        )(y, st, gprev, beprev, w, b), (g, be)

    _, _, g_c, be_c = _pb(params["first_pn"][0])
    (y, st), (g_c, be_c) = mid_layer(y, st, params["first_pn"][1], g_c, be_c)
    (y, st), (g_c, be_c) = mid_layer(y, st, params["first_pn"][2], g_c, be_c)

    nwb = 32 // nb
    idx_flat = idx.reshape(-1)
    f1 = pl.pallas_call(
        functools.partial(_normf_k, cnt=float(cntN)),
        grid=(nb, t),
        in_specs=[tile((1, c, KS, pn), 3), full((nb, 2, c)),
                  full(g_c.shape), full(be_c.shape)],
        out_specs=tile((1, KS, pn, c), 2),
        out_shape=jax.ShapeDtypeStruct((nb, KS, n, c), F32),
    )(y, st, g_c, be_c)
    f1m = _sc_segmax(f1.reshape(-1), idx_flat, nb, c, n, m)

    w4, b4, g4, be4 = _pb(params["second_pn"][0])
    c2 = w4.shape[0]
    y, st = pl.pallas_call(
        functools.partial(_l4_k, pn=pn, m=m, cin=c, cout=c2),
        grid=(nb, t),
        in_specs=[tile((1, KS, pn, c), 2), tile((1, KS, pn), 2),
                  perb((1, nwb, m, c)), full(w4.shape), full(b4.shape)],
        out_specs=[tile((1, c2, KS, pn), 3), perb((1, 2, c2))],
        out_shape=[jax.ShapeDtypeStruct((nb, c2, KS, n), F32),
                   jax.ShapeDtypeStruct((nb, 2, c2), F32)],
    )(f1, idx, f1m, w4, b4)

    (y, st), (g_c, be_c) = mid_layer(y, st, params["second_pn"][1], g4, be4)

    f2 = pl.pallas_call(
        functools.partial(_normf_k, cnt=float(cntN)),
        grid=(nb, t),
        in_specs=[tile((1, c2, KS, pn), 3), full((nb, 2, c2)),
                  full(g_c.shape), full(be_c.shape)],
        out_specs=tile((1, KS, pn, c2), 2),
        out_shape=jax.ShapeDtypeStruct((nb, KS, n, c2), F32),
    )(y, st, g_c, be_c)
    f2p = _sc_segmax(f2.reshape(-1), idx_flat, nb, c2, n, m)

    kf = [_pb(p) for p in params["knn_first"]]
    ck = kf[0][0].shape[0]
    g_nodes, f2m = pl.pallas_call(
        functools.partial(_knn_k, nb=nb, m=m),
        out_shape=[jax.ShapeDtypeStruct((nb, ck, m), F32),
                   jax.ShapeDtypeStruct((nb, c2, m), F32)],
    )(sums, cnt, f2p, *kf[0], *kf[1], *kf[2])

    ksnd = [_pb(p) for p in params["knn_second"]]
    m1 = _pb(params["mlp1"][0])
    m2 = _pb(params["mlp2"][0])
    w3p = params["mlp3"][0]["W"]
    b3p = params["mlp3"][0]["b"].reshape(-1, 1)
    cm_out, kp, sg = pl.pallas_call(
        functools.partial(_tail_k, nb=nb, m=m, c2=c2),
        out_shape=[jax.ShapeDtypeStruct((nb, 3, m), F32),
                   jax.ShapeDtypeStruct((nb, 3, m), F32),
                   jax.ShapeDtypeStruct((nb, m), F32)],
    )(g_nodes, f2m, sums, cnt, *ksnd[0], *ksnd[1], *m1, *m2, w3p, b3p)

    return (cm_out, kp, sg)
